# Initial kernel scaffold; baseline (speedup 1.0000x reference)
import jax
import jax.numpy as jnp
from jax.experimental import pallas as pl


def kernel(logit, target, mask):
    raise NotImplementedError("write your pallas kernel here")


# speedup vs baseline: 29.7494x; 29.7494x over previous
import jax
import jax.numpy as jnp
from jax.experimental import pallas as pl
from jax.experimental.pallas import tpu as pltpu

_NR1 = 26
_ROWS = 16640
_COLS = 128
_M_POS = -1.0
_M_NEG = -2.0
_GAMMA = 1.0


def _orderkey(v):
    i = jax.lax.bitcast_convert_type(v, jnp.int32)
    return jnp.where(i < 0, i ^ jnp.int32(0x7FFFFFFF), i)


def _values(lg, tg, mk):
    mf = mk.astype(jnp.float32)
    vpos = lg * mf * (tg == 1).astype(jnp.float32)
    vpos = jnp.where(vpos == 0.0, jnp.float32(1.0), vpos)
    vneg = lg * mf * (tg == 0).astype(jnp.float32)
    return vpos, vneg


def _fpos(v):
    return (jnp.log(1.0 + jnp.exp(_GAMMA * (_M_POS - v)))
            + jnp.log(1.0 + jnp.exp(_GAMMA * (-100.0 + v))))


def _fneg(v):
    return (jnp.log(1.0 + jnp.exp(_GAMMA * (_M_NEG + v)))
            + jnp.log(1.0 + jnp.exp(_GAMMA * (-100.0 - v))))


_N_CHUNK = 8
_CH = _ROWS // _N_CHUNK


def _body(lg_ref, tg_ref, mk_ref, out_ref, kpos_ref, kneg_ref):
    def p1(c, k_acc):
        sl = pl.ds(c * _CH, _CH)
        lg = lg_ref[sl, :]
        tg = tg_ref[sl, :]
        mk = mk_ref[sl, :]
        vpos, vneg = _values(lg, tg, mk)
        kpos_ref[sl, :] = _orderkey(vpos)
        kneg_ref[sl, :] = ~_orderkey(vneg)
        return k_acc + jnp.sum((tg == 1).astype(jnp.int32))

    k = jax.lax.fori_loop(0, _N_CHUNK, p1, jnp.int32(0))

    def bisect(key_ref):
        def step(_, lohi):
            lo, hi = lohi
            mid = (lo & hi) + ((lo ^ hi) >> 1)
            def cchunk(c, acc):
                blk = key_ref[pl.ds(c * _CH, _CH), :]
                return acc + jnp.sum((blk <= mid).astype(jnp.int32))
            cnt = jax.lax.fori_loop(0, _N_CHUNK, cchunk, jnp.int32(0))
            ge = cnt >= k
            return (jnp.where(ge, lo, mid + 1), jnp.where(ge, mid, hi))
        lo, _ = jax.lax.fori_loop(
            0, 32, step,
            (jnp.int32(-(2 ** 31)), jnp.int32(2 ** 31 - 1)))
        return lo

    kp_star = bisect(kpos_ref)
    kn_star = bisect(kneg_ref)

    def p3(c, carry):
        cnt_ltp, cnt_ltn, fsum_ltp, fsum_ltn, fp_at, fn_at = carry
        sl = pl.ds(c * _CH, _CH)
        vpos, vneg = _values(lg_ref[sl, :], tg_ref[sl, :], mk_ref[sl, :])
        fpos = _fpos(vpos)
        fneg = _fneg(vneg)
        kp = kpos_ref[sl, :]
        kn = kneg_ref[sl, :]
        ltp = kp < kp_star
        ltn = kn < kn_star
        cnt_ltp = cnt_ltp + jnp.sum(ltp.astype(jnp.int32))
        cnt_ltn = cnt_ltn + jnp.sum(ltn.astype(jnp.int32))
        fsum_ltp = fsum_ltp + jnp.sum(jnp.where(ltp, fpos, 0.0))
        fsum_ltn = fsum_ltn + jnp.sum(jnp.where(ltn, fneg, 0.0))
        fp_at = jnp.minimum(fp_at, jnp.min(
            jnp.where(kp == kp_star, fpos, jnp.inf)))
        fn_at = jnp.minimum(fn_at, jnp.min(
            jnp.where(kn == kn_star, fneg, jnp.inf)))
        return cnt_ltp, cnt_ltn, fsum_ltp, fsum_ltn, fp_at, fn_at

    cnt_ltp, cnt_ltn, fsum_ltp, fsum_ltn, fp_at, fn_at = jax.lax.fori_loop(
        0, _N_CHUNK, p3,
        (jnp.int32(0), jnp.int32(0), jnp.float32(0.0), jnp.float32(0.0),
         jnp.float32(jnp.inf), jnp.float32(jnp.inf)))

    part1 = fsum_ltp + (k - cnt_ltp).astype(jnp.float32) * fp_at
    part2 = fsum_ltn + (k - cnt_ltn).astype(jnp.float32) * fn_at
    out_ref[0, 0] = (part1 + part2) / k.astype(jnp.float32)


def kernel(logit, target, mask):
    lg = logit.reshape(_ROWS, _COLS)
    tg = target.reshape(_ROWS, _COLS)
    mk = jnp.broadcast_to(mask[:, :, None], mask.shape + (_NR1,)).reshape(
        _ROWS, _COLS)
    out = pl.pallas_call(
        _body,
        out_shape=jax.ShapeDtypeStruct((1, 1), jnp.float32),
        out_specs=pl.BlockSpec(memory_space=pltpu.SMEM),
        scratch_shapes=[
            pltpu.VMEM((_ROWS, _COLS), jnp.int32),
            pltpu.VMEM((_ROWS, _COLS), jnp.int32),
        ],
    )(lg, tg, mk)
    return out.reshape(())


# speedup vs baseline: 32.2028x; 1.0825x over previous
import jax
import jax.numpy as jnp
from jax.experimental import pallas as pl
from jax.experimental.pallas import tpu as pltpu

_NR1 = 26
_ROWS = 16640
_COLS = 128
_M_POS = -1.0
_M_NEG = -2.0
_GAMMA = 1.0


def _orderkey(v):
    i = jax.lax.bitcast_convert_type(v, jnp.int32)
    return jnp.where(i < 0, i ^ jnp.int32(0x7FFFFFFF), i)


def _values(lg, tg, mk):
    mf = mk.astype(jnp.float32)
    vpos = lg * mf * (tg == 1).astype(jnp.float32)
    vpos = jnp.where(vpos == 0.0, jnp.float32(1.0), vpos)
    vneg = lg * mf * (tg == 0).astype(jnp.float32)
    return vpos, vneg


def _fpos(v):
    return (jnp.log(1.0 + jnp.exp(_GAMMA * (_M_POS - v)))
            + jnp.log(1.0 + jnp.exp(_GAMMA * (-100.0 + v))))


def _fneg(v):
    return (jnp.log(1.0 + jnp.exp(_GAMMA * (_M_NEG + v)))
            + jnp.log(1.0 + jnp.exp(_GAMMA * (-100.0 - v))))


_N_CHUNK = 8
_CH = _ROWS // _N_CHUNK


def _body(lg_ref, tg_ref, mk_ref, out_ref, kpos_ref, kneg_ref):
    def p1(c, k_acc):
        sl = pl.ds(c * _CH, _CH)
        lg = lg_ref[sl, :]
        tg = tg_ref[sl, :]
        mk = mk_ref[sl, :]
        vpos, vneg = _values(lg, tg, mk)
        kpos_ref[sl, :] = _orderkey(vpos)
        kneg_ref[sl, :] = ~_orderkey(vneg)
        return k_acc + jnp.sum((tg == 1).astype(jnp.int32))

    k = jax.lax.fori_loop(0, _N_CHUNK, p1, jnp.int32(0))

    sign = jnp.int32(-(2 ** 31))

    def digit_select(key_ref):
        prefix = jnp.int32(0)
        for s, w in [(29, 3)] + [(29 - 3 * i, 3) for i in range(1, 10)] + [(0, 2)]:
            nd = (1 << w) - 1
            low_ones = jnp.int32((1 << s) - 1)

            def _i32(v):
                v &= 0xFFFFFFFF
                return jnp.int32(v - (1 << 32) if v >= (1 << 31) else v)

            thr = [((prefix | _i32(d << s)) | low_ones) ^ sign
                   for d in range(nd)]

            def cchunk(c, acc):
                blk = key_ref[pl.ds(c * _CH, _CH), :]
                return tuple(
                    a + jnp.sum((blk <= t).astype(jnp.int32))
                    for a, t in zip(acc, thr))
            cnts = jax.lax.fori_loop(
                0, _N_CHUNK, cchunk, tuple(jnp.int32(0) for _ in range(nd)))

            dstar = jnp.int32(nd)
            for d in range(nd - 1, -1, -1):
                dstar = jnp.where(cnts[d] >= k, jnp.int32(d), dstar)
            prefix = prefix | (dstar << s)
        return prefix ^ sign

    kp_star = digit_select(kpos_ref)
    kn_star = digit_select(kneg_ref)

    def p3(c, carry):
        cnt_ltp, cnt_ltn, fsum_ltp, fsum_ltn, fp_at, fn_at = carry
        sl = pl.ds(c * _CH, _CH)
        vpos, vneg = _values(lg_ref[sl, :], tg_ref[sl, :], mk_ref[sl, :])
        fpos = _fpos(vpos)
        fneg = _fneg(vneg)
        kp = kpos_ref[sl, :]
        kn = kneg_ref[sl, :]
        ltp = kp < kp_star
        ltn = kn < kn_star
        cnt_ltp = cnt_ltp + jnp.sum(ltp.astype(jnp.int32))
        cnt_ltn = cnt_ltn + jnp.sum(ltn.astype(jnp.int32))
        fsum_ltp = fsum_ltp + jnp.sum(jnp.where(ltp, fpos, 0.0))
        fsum_ltn = fsum_ltn + jnp.sum(jnp.where(ltn, fneg, 0.0))
        fp_at = jnp.minimum(fp_at, jnp.min(
            jnp.where(kp == kp_star, fpos, jnp.inf)))
        fn_at = jnp.minimum(fn_at, jnp.min(
            jnp.where(kn == kn_star, fneg, jnp.inf)))
        return cnt_ltp, cnt_ltn, fsum_ltp, fsum_ltn, fp_at, fn_at

    cnt_ltp, cnt_ltn, fsum_ltp, fsum_ltn, fp_at, fn_at = jax.lax.fori_loop(
        0, _N_CHUNK, p3,
        (jnp.int32(0), jnp.int32(0), jnp.float32(0.0), jnp.float32(0.0),
         jnp.float32(jnp.inf), jnp.float32(jnp.inf)))

    part1 = fsum_ltp + (k - cnt_ltp).astype(jnp.float32) * fp_at
    part2 = fsum_ltn + (k - cnt_ltn).astype(jnp.float32) * fn_at
    out_ref[0, 0] = (part1 + part2) / k.astype(jnp.float32)


def kernel(logit, target, mask):
    lg = logit.reshape(_ROWS, _COLS)
    tg = target.reshape(_ROWS, _COLS)
    mk = jnp.broadcast_to(mask[:, :, None], mask.shape + (_NR1,)).reshape(
        _ROWS, _COLS)
    out = pl.pallas_call(
        _body,
        out_shape=jax.ShapeDtypeStruct((1, 1), jnp.float32),
        out_specs=pl.BlockSpec(memory_space=pltpu.SMEM),
        scratch_shapes=[
            pltpu.VMEM((_ROWS, _COLS), jnp.int32),
            pltpu.VMEM((_ROWS, _COLS), jnp.int32),
        ],
    )(lg, tg, mk)
    return out.reshape(())


# speedup vs baseline: 36.0863x; 1.1206x over previous
import jax
import jax.numpy as jnp
from jax.experimental import pallas as pl
from jax.experimental.pallas import tpu as pltpu

_NR1 = 26
_ROWS = 16640
_COLS = 128
_M_POS = -1.0
_M_NEG = -2.0
_GAMMA = 1.0


def _orderkey(v):
    i = jax.lax.bitcast_convert_type(v, jnp.int32)
    return jnp.where(i < 0, i ^ jnp.int32(0x7FFFFFFF), i)


def _invkey(key):
    bits = jnp.where(key < 0, key ^ jnp.int32(0x7FFFFFFF), key)
    return jax.lax.bitcast_convert_type(bits, jnp.float32)


def _values(lg, tg):
    vpos = lg * (tg == 1).astype(jnp.float32)
    vpos = jnp.where(vpos == 0.0, jnp.float32(1.0), vpos)
    vneg = lg * (tg == 0).astype(jnp.float32)
    return vpos, vneg


def _fpos(v):
    return (jnp.log(1.0 + jnp.exp(_GAMMA * (_M_POS - v)))
            + jnp.log(1.0 + jnp.exp(_GAMMA * (-100.0 + v))))


def _fneg(v):
    return (jnp.log(1.0 + jnp.exp(_GAMMA * (_M_NEG + v)))
            + jnp.log(1.0 + jnp.exp(_GAMMA * (-100.0 - v))))


_N_CHUNK = 8
_CH = _ROWS // _N_CHUNK


def _body(lg_ref, tg_ref, out_ref, kpos_ref, kneg_ref):
    def p1(c, k_acc):
        sl = pl.ds(c * _CH, _CH)
        lg = lg_ref[sl, :]
        tg = tg_ref[sl, :]
        vpos, vneg = _values(lg, tg)
        kpos_ref[sl, :] = _orderkey(vpos)
        kneg_ref[sl, :] = ~_orderkey(vneg)
        return (k_acc + jnp.sum((tg == 1).astype(jnp.int32))
                + jnp.sum((tg == -2).astype(jnp.int32)))

    k = jax.lax.fori_loop(0, _N_CHUNK, p1, jnp.int32(0))

    sign = jnp.int32(-(2 ** 31))

    def digit_select(key_ref):
        prefix = jnp.int32(0)
        for s, w in [(29, 3)] + [(29 - 3 * i, 3) for i in range(1, 10)] + [(0, 2)]:
            nd = (1 << w) - 1
            low_ones = jnp.int32((1 << s) - 1)

            def _i32(v):
                v &= 0xFFFFFFFF
                return jnp.int32(v - (1 << 32) if v >= (1 << 31) else v)

            thr = [((prefix | _i32(d << s)) | low_ones) ^ sign
                   for d in range(nd)]

            def cchunk(c, acc):
                blk = key_ref[pl.ds(c * _CH, _CH), :]
                return tuple(
                    a + jnp.sum((blk <= t).astype(jnp.int32))
                    for a, t in zip(acc, thr))
            cnts = jax.lax.fori_loop(
                0, _N_CHUNK, cchunk, tuple(jnp.int32(0) for _ in range(nd)))

            dstar = jnp.int32(nd)
            for d in range(nd - 1, -1, -1):
                dstar = jnp.where(cnts[d] >= k, jnp.int32(d), dstar)
            prefix = prefix | (dstar << s)
        return prefix ^ sign

    kp_star = digit_select(kpos_ref)
    kn_star = digit_select(kneg_ref)

    def p3(c, carry):
        cnt_ltp, cnt_ltn, fsum_ltp, fsum_ltn, fp_at, fn_at = carry
        sl = pl.ds(c * _CH, _CH)
        kp = kpos_ref[sl, :]
        kn = kneg_ref[sl, :]
        vpos = _invkey(kp)
        vneg = _invkey(~kn)
        fpos = _fpos(vpos)
        fneg = _fneg(vneg)
        ltp = kp < kp_star
        ltn = kn < kn_star
        cnt_ltp = cnt_ltp + jnp.sum(ltp.astype(jnp.int32))
        cnt_ltn = cnt_ltn + jnp.sum(ltn.astype(jnp.int32))
        fsum_ltp = fsum_ltp + jnp.sum(jnp.where(ltp, fpos, 0.0))
        fsum_ltn = fsum_ltn + jnp.sum(jnp.where(ltn, fneg, 0.0))
        fp_at = jnp.minimum(fp_at, jnp.min(
            jnp.where(kp == kp_star, fpos, jnp.inf)))
        fn_at = jnp.minimum(fn_at, jnp.min(
            jnp.where(kn == kn_star, fneg, jnp.inf)))
        return cnt_ltp, cnt_ltn, fsum_ltp, fsum_ltn, fp_at, fn_at

    cnt_ltp, cnt_ltn, fsum_ltp, fsum_ltn, fp_at, fn_at = jax.lax.fori_loop(
        0, _N_CHUNK, p3,
        (jnp.int32(0), jnp.int32(0), jnp.float32(0.0), jnp.float32(0.0),
         jnp.float32(jnp.inf), jnp.float32(jnp.inf)))

    part1 = fsum_ltp + (k - cnt_ltp).astype(jnp.float32) * fp_at
    part2 = fsum_ltn + (k - cnt_ltn).astype(jnp.float32) * fn_at
    out_ref[0, 0] = (part1 + part2) / k.astype(jnp.float32)


def kernel(logit, target, mask):
    lg = logit.reshape(_ROWS, _COLS)
    m3 = mask[:, :, None] != 0
    tgf = jnp.where(m3, target, jnp.where(target == 1, -2, -1))
    tg = tgf.reshape(_ROWS, _COLS)
    out = pl.pallas_call(
        _body,
        out_shape=jax.ShapeDtypeStruct((1, 1), jnp.float32),
        out_specs=pl.BlockSpec(memory_space=pltpu.SMEM),
        scratch_shapes=[
            pltpu.VMEM((_ROWS, _COLS), jnp.int32),
            pltpu.VMEM((_ROWS, _COLS), jnp.int32),
        ],
    )(lg, tg)
    return out.reshape(())


# speedup vs baseline: 76.1143x; 2.1092x over previous
import jax
import jax.numpy as jnp
from jax.experimental import pallas as pl
from jax.experimental.pallas import tpu as pltpu

_NR1 = 26
_ROWS = 16640
_COLS = 128
_M_POS = -1.0
_M_NEG = -2.0
_GAMMA = 1.0


def _orderkey(v):
    i = jax.lax.bitcast_convert_type(v, jnp.int32)
    return jnp.where(i < 0, i ^ jnp.int32(0x7FFFFFFF), i)


def _invkey(key):
    bits = jnp.where(key < 0, key ^ jnp.int32(0x7FFFFFFF), key)
    return jax.lax.bitcast_convert_type(bits, jnp.float32)


def _values(lg, tg):
    vpos = lg * (tg == 1).astype(jnp.float32)
    vpos = jnp.where(vpos == 0.0, jnp.float32(1.0), vpos)
    vneg = lg * (tg == 0).astype(jnp.float32)
    return vpos, vneg


def _fpos(v):
    return (jnp.log(1.0 + jnp.exp(_GAMMA * (_M_POS - v)))
            + jnp.log(1.0 + jnp.exp(_GAMMA * (-100.0 + v))))


def _fneg(v):
    return (jnp.log(1.0 + jnp.exp(_GAMMA * (_M_NEG + v)))
            + jnp.log(1.0 + jnp.exp(_GAMMA * (-100.0 - v))))


_N_CHUNK = 8
_CH = _ROWS // _N_CHUNK


def _body(lg_ref, tg_ref, out_ref, kpos_ref, kneg_ref):
    _C1P = jnp.int32(0x3F800000)

    def p1(c, acc):
        k_acc, ltp, eqp, ltn, eqna, eqnb = acc
        sl = pl.ds(c * _CH, _CH)
        lg = lg_ref[sl, :]
        tg = tg_ref[sl, :]
        vpos, vneg = _values(lg, tg)
        kp = _orderkey(vpos)
        kn = ~_orderkey(vneg)
        kpos_ref[sl, :] = kp
        kneg_ref[sl, :] = kn
        i32 = jnp.int32
        return (k_acc + jnp.sum((tg == 1).astype(i32))
                + jnp.sum((tg == -2).astype(i32)),
                ltp + jnp.sum((kp < _C1P).astype(i32)),
                eqp + jnp.sum((kp == _C1P).astype(i32)),
                ltn + jnp.sum((kn < -1).astype(i32)),
                eqna + jnp.sum((kn == -1).astype(i32)),
                eqnb + jnp.sum((kn == 0).astype(i32)))

    z = jnp.int32(0)
    k, ltp, eqp, ltn, eqna, eqnb = jax.lax.fori_loop(
        0, _N_CHUNK, p1, (z, z, z, z, z, z))

    sign = jnp.int32(-(2 ** 31))

    def digit_select(key_ref):
        prefix = jnp.int32(0)
        for s, w in [(29, 3)] + [(29 - 3 * i, 3) for i in range(1, 10)] + [(0, 2)]:
            nd = (1 << w) - 1
            low_ones = jnp.int32((1 << s) - 1)

            def _i32(v):
                v &= 0xFFFFFFFF
                return jnp.int32(v - (1 << 32) if v >= (1 << 31) else v)

            thr = [((prefix | _i32(d << s)) | low_ones) ^ sign
                   for d in range(nd)]

            def cchunk(c, acc):
                blk = key_ref[pl.ds(c * _CH, _CH), :]
                return tuple(
                    a + jnp.sum((blk <= t).astype(jnp.int32))
                    for a, t in zip(acc, thr))
            cnts = jax.lax.fori_loop(
                0, _N_CHUNK, cchunk, tuple(jnp.int32(0) for _ in range(nd)))

            dstar = jnp.int32(nd)
            for d in range(nd - 1, -1, -1):
                dstar = jnp.where(cnts[d] >= k, jnp.int32(d), dstar)
            prefix = prefix | (dstar << s)
        return prefix ^ sign

    fast_p = (ltp < k) & (k <= ltp + eqp)
    kp_star = jax.lax.cond(
        fast_p, lambda: _C1P, lambda: digit_select(kpos_ref))

    fast_n = (ltn < k) & (k <= ltn + eqna + eqnb)
    fastval_n = jnp.where(k <= ltn + eqna, jnp.int32(-1), jnp.int32(0))
    kn_star = jax.lax.cond(
        fast_n, lambda: fastval_n, lambda: digit_select(kneg_ref))

    def p3(c, carry):
        cnt_ltp, cnt_ltn, fsum_ltp, fsum_ltn, fp_at, fn_at = carry
        sl = pl.ds(c * _CH, _CH)
        kp = kpos_ref[sl, :]
        kn = kneg_ref[sl, :]
        vpos = _invkey(kp)
        vneg = _invkey(~kn)
        fpos = _fpos(vpos)
        fneg = _fneg(vneg)
        ltp = kp < kp_star
        ltn = kn < kn_star
        cnt_ltp = cnt_ltp + jnp.sum(ltp.astype(jnp.int32))
        cnt_ltn = cnt_ltn + jnp.sum(ltn.astype(jnp.int32))
        fsum_ltp = fsum_ltp + jnp.sum(jnp.where(ltp, fpos, 0.0))
        fsum_ltn = fsum_ltn + jnp.sum(jnp.where(ltn, fneg, 0.0))
        fp_at = jnp.minimum(fp_at, jnp.min(
            jnp.where(kp == kp_star, fpos, jnp.inf)))
        fn_at = jnp.minimum(fn_at, jnp.min(
            jnp.where(kn == kn_star, fneg, jnp.inf)))
        return cnt_ltp, cnt_ltn, fsum_ltp, fsum_ltn, fp_at, fn_at

    cnt_ltp, cnt_ltn, fsum_ltp, fsum_ltn, fp_at, fn_at = jax.lax.fori_loop(
        0, _N_CHUNK, p3,
        (jnp.int32(0), jnp.int32(0), jnp.float32(0.0), jnp.float32(0.0),
         jnp.float32(jnp.inf), jnp.float32(jnp.inf)))

    part1 = fsum_ltp + (k - cnt_ltp).astype(jnp.float32) * fp_at
    part2 = fsum_ltn + (k - cnt_ltn).astype(jnp.float32) * fn_at
    out_ref[0, 0] = (part1 + part2) / k.astype(jnp.float32)


def kernel(logit, target, mask):
    lg = logit.reshape(_ROWS, _COLS)
    m3 = mask[:, :, None] != 0
    tgf = jnp.where(m3, target, jnp.where(target == 1, -2, -1))
    tg = tgf.reshape(_ROWS, _COLS)
    out = pl.pallas_call(
        _body,
        out_shape=jax.ShapeDtypeStruct((1, 1), jnp.float32),
        out_specs=pl.BlockSpec(memory_space=pltpu.SMEM),
        scratch_shapes=[
            pltpu.VMEM((_ROWS, _COLS), jnp.int32),
            pltpu.VMEM((_ROWS, _COLS), jnp.int32),
        ],
    )(lg, tg)
    return out.reshape(())


# speedup vs baseline: 223.9917x; 2.9428x over previous
import jax
import jax.numpy as jnp
from jax.experimental import pallas as pl
from jax.experimental.pallas import tpu as pltpu

_D0 = 26
_D1 = 20
_D2 = 4096
_M_POS = -1.0
_M_NEG = -2.0
_GAMMA = 1.0

_N_CHUNK = 13
_CH = _D0 // _N_CHUNK


def _orderkey(v):
    i = jax.lax.bitcast_convert_type(v, jnp.int32)
    return jnp.where(i < 0, i ^ jnp.int32(0x7FFFFFFF), i)


def _invkey(key):
    bits = jnp.where(key < 0, key ^ jnp.int32(0x7FFFFFFF), key)
    return jax.lax.bitcast_convert_type(bits, jnp.float32)


def _values(lg, tg, mf):
    vpos = lg * mf * (tg == 1).astype(jnp.float32)
    vpos = jnp.where(vpos == 0.0, jnp.float32(1.0), vpos)
    vneg = lg * mf * (tg == 0).astype(jnp.float32)
    return vpos, vneg


def _fpos(v):
    return (jnp.log(1.0 + jnp.exp(_GAMMA * (_M_POS - v)))
            + jnp.log(1.0 + jnp.exp(_GAMMA * (-100.0 + v))))


def _fneg(v):
    return (jnp.log(1.0 + jnp.exp(_GAMMA * (_M_NEG + v)))
            + jnp.log(1.0 + jnp.exp(_GAMMA * (-100.0 - v))))


def _body(lg_ref, tg_ref, mk_ref, out_ref, kpos_ref, kneg_ref):
    _C1P = jnp.int32(0x3F800000)
    mf = mk_ref[...].astype(jnp.float32)[None]

    def p1(c, acc):
        k_acc, ltp, eqp, ltn, eqna, eqnb = acc
        sl = pl.ds(c * _CH, _CH)
        lg = lg_ref[sl]
        tg = tg_ref[sl]
        vpos, vneg = _values(lg, tg, mf)
        kp = _orderkey(vpos)
        kn = ~_orderkey(vneg)
        kpos_ref[sl] = kp
        kneg_ref[sl] = kn
        i32 = jnp.int32
        return (k_acc + jnp.sum((tg == 1).astype(i32)),
                ltp + jnp.sum((kp < _C1P).astype(i32)),
                eqp + jnp.sum((kp == _C1P).astype(i32)),
                ltn + jnp.sum((kn < -1).astype(i32)),
                eqna + jnp.sum((kn == -1).astype(i32)),
                eqnb + jnp.sum((kn == 0).astype(i32)))

    z = jnp.int32(0)
    k, ltp, eqp, ltn, eqna, eqnb = jax.lax.fori_loop(
        0, _N_CHUNK, p1, (z, z, z, z, z, z))

    sign = jnp.int32(-(2 ** 31))

    def digit_select(key_ref):
        prefix = jnp.int32(0)
        for s, w in [(29, 3)] + [(29 - 3 * i, 3) for i in range(1, 10)] + [(0, 2)]:
            nd = (1 << w) - 1
            low_ones = jnp.int32((1 << s) - 1)

            def _i32(v):
                v &= 0xFFFFFFFF
                return jnp.int32(v - (1 << 32) if v >= (1 << 31) else v)

            thr = [((prefix | _i32(d << s)) | low_ones) ^ sign
                   for d in range(nd)]

            def cchunk(c, acc):
                blk = key_ref[pl.ds(c * _CH, _CH)]
                return tuple(
                    a + jnp.sum((blk <= t).astype(jnp.int32))
                    for a, t in zip(acc, thr))
            cnts = jax.lax.fori_loop(
                0, _N_CHUNK, cchunk, tuple(jnp.int32(0) for _ in range(nd)))

            dstar = jnp.int32(nd)
            for d in range(nd - 1, -1, -1):
                dstar = jnp.where(cnts[d] >= k, jnp.int32(d), dstar)
            prefix = prefix | (dstar << s)
        return prefix ^ sign

    fast_p = (ltp < k) & (k <= ltp + eqp)
    kp_star = jax.lax.cond(
        fast_p, lambda: _C1P, lambda: digit_select(kpos_ref))

    fast_n = (ltn < k) & (k <= ltn + eqna + eqnb)
    fastval_n = jnp.where(k <= ltn + eqna, jnp.int32(-1), jnp.int32(0))
    kn_star = jax.lax.cond(
        fast_n, lambda: fastval_n, lambda: digit_select(kneg_ref))

    def p3(c, carry):
        cnt_ltp, cnt_ltn, fsum_ltp, fsum_ltn, fp_at, fn_at = carry
        sl = pl.ds(c * _CH, _CH)
        kp = kpos_ref[sl]
        kn = kneg_ref[sl]
        vpos = _invkey(kp)
        vneg = _invkey(~kn)
        fpos = _fpos(vpos)
        fneg = _fneg(vneg)
        ltp_m = kp < kp_star
        ltn_m = kn < kn_star
        cnt_ltp = cnt_ltp + jnp.sum(ltp_m.astype(jnp.int32))
        cnt_ltn = cnt_ltn + jnp.sum(ltn_m.astype(jnp.int32))
        fsum_ltp = fsum_ltp + jnp.sum(jnp.where(ltp_m, fpos, 0.0))
        fsum_ltn = fsum_ltn + jnp.sum(jnp.where(ltn_m, fneg, 0.0))
        fp_at = jnp.minimum(fp_at, jnp.min(
            jnp.where(kp == kp_star, fpos, jnp.inf)))
        fn_at = jnp.minimum(fn_at, jnp.min(
            jnp.where(kn == kn_star, fneg, jnp.inf)))
        return cnt_ltp, cnt_ltn, fsum_ltp, fsum_ltn, fp_at, fn_at

    cnt_ltp, cnt_ltn, fsum_ltp, fsum_ltn, fp_at, fn_at = jax.lax.fori_loop(
        0, _N_CHUNK, p3,
        (jnp.int32(0), jnp.int32(0), jnp.float32(0.0), jnp.float32(0.0),
         jnp.float32(jnp.inf), jnp.float32(jnp.inf)))

    part1 = fsum_ltp + (k - cnt_ltp).astype(jnp.float32) * fp_at
    part2 = fsum_ltn + (k - cnt_ltn).astype(jnp.float32) * fn_at
    out_ref[0, 0] = (part1 + part2) / k.astype(jnp.float32)


def kernel(logit, target, mask):
    lg = jnp.transpose(logit, (2, 1, 0))
    tg = jnp.transpose(target, (2, 1, 0))
    mk = jnp.transpose(mask, (1, 0))
    out = pl.pallas_call(
        _body,
        out_shape=jax.ShapeDtypeStruct((1, 1), jnp.float32),
        out_specs=pl.BlockSpec(memory_space=pltpu.SMEM),
        scratch_shapes=[
            pltpu.VMEM((_D0, _D1, _D2), jnp.int32),
            pltpu.VMEM((_D0, _D1, _D2), jnp.int32),
        ],
    )(lg, tg, mk)
    return out.reshape(())
